# Initial kernel scaffold; baseline (speedup 1.0000x reference)
import jax
import jax.numpy as jnp
from jax.experimental import pallas as pl


def kernel(pred, gt_boxes, gt_classes, anchors):
    raise NotImplementedError("write your pallas kernel here")


# speedup vs baseline: 12.0416x; 12.0416x over previous
import functools

import jax
import jax.numpy as jnp
from jax.experimental import pallas as pl
from jax.experimental.pallas import tpu as pltpu

_NEG_RATIO = 3.0
_IOU_THR = 0.5
_ALPHA = 1.0

_R = 72
_L = 128
_NBP = _R * _L


def _row_body(pred_ref, anc_ref, gtb_ref, gtc_ref, out_ref, acc_ref, *, nb, nc, nbatch):
    r = pl.program_id(0)
    f32 = jnp.float32

    ax = anc_ref[0]
    ay = anc_ref[1]
    aw = anc_ref[2]
    ah = anc_ref[3]
    ax1 = ax - aw / 2
    ay1 = ay - ah / 2
    ax2 = ax + aw / 2
    ay2 = ay + ah / 2
    area_a = (ax2 - ax1) * (ay2 - ay1)

    row_ids = jax.lax.broadcasted_iota(jnp.int32, (_R, _L), 0)
    lane_ids = jax.lax.broadcasted_iota(jnp.int32, (_R, _L), 1)
    ids = row_ids * _L + lane_ids
    valid = ids < nb

    g_cx = [gtb_ref[0, g, 0] for g in range(8)]
    g_cy = [gtb_ref[0, g, 1] for g in range(8)]
    g_w = [gtb_ref[0, g, 2] for g in range(8)]
    g_h = [gtb_ref[0, g, 3] for g in range(8)]
    g_cls = [(gtc_ref[0, 0, g] + 1).astype(f32) for g in range(8)]

    best_ov = None
    best_idx = None
    bp = []
    for g in range(8):
        gx1 = g_cx[g] - g_w[g] / 2
        gy1 = g_cy[g] - g_h[g] / 2
        gx2 = g_cx[g] + g_w[g] / 2
        gy2 = g_cy[g] + g_h[g] / 2
        garea = (gx2 - gx1) * (gy2 - gy1)
        iw = jnp.maximum(jnp.minimum(gx2, ax2) - jnp.maximum(gx1, ax1), 0.0)
        ih = jnp.maximum(jnp.minimum(gy2, ay2) - jnp.maximum(gy1, ay1), 0.0)
        inter = iw * ih
        iou = inter / (garea + area_a - inter + 1e-12)
        if g == 0:
            best_ov = iou
            best_idx = jnp.zeros((_R, _L), jnp.int32)
        else:
            upd = iou > best_ov
            best_ov = jnp.where(upd, iou, best_ov)
            best_idx = jnp.where(upd, g, best_idx)
        m = jnp.max(iou)
        cand = jnp.where(iou == m, ids, _NBP)
        bp.append(jnp.min(cand))

    def pick(vals):
        v = jnp.full((_R, _L), vals[0], f32)
        for g in range(1, 8):
            v = jnp.where(best_idx == g, vals[g], v)
        return v

    cx_p = pick(g_cx)
    cy_p = pick(g_cy)
    w_p = pick(g_w)
    h_p = pick(g_h)
    cls_p = pick(g_cls)

    over = best_ov > _IOU_THR
    match = over
    for g in range(8):
        fm = ids == bp[g]
        cx_p = jnp.where(fm, g_cx[g], cx_p)
        cy_p = jnp.where(fm, g_cy[g], cy_p)
        w_p = jnp.where(fm, g_w[g], w_p)
        h_p = jnp.where(fm, g_h[g], h_p)
        cls_p = jnp.where(fm, g_cls[g], cls_p)
        match = jnp.logical_or(match, fm)

    law = jnp.log(aw)
    lah = jnp.log(ah)
    w_safe = jnp.where(match, w_p, 1.0)
    h_safe = jnp.where(match, h_p, 1.0)
    zero = jnp.zeros((_R, _L), f32)
    t0 = jnp.where(match, (cx_p - ax) / aw, zero)
    t1 = jnp.where(match, (cy_p - ay) / ah, zero)
    t2 = jnp.where(match, jnp.log(w_safe) - law, zero)
    t3 = jnp.where(match, jnp.log(h_safe) - lah, zero)
    cls_t = jnp.where(match, cls_p, zero)

    pos = (cls_t > 0).astype(f32)
    np_row = jnp.sum(pos)

    locs = (t0, t1, t2, t3)
    sl1 = zero
    for c in range(4):
        d = pred_ref[0, c] - locs[c]
        ad = jnp.abs(d)
        sl1 = sl1 + jnp.where(ad < 1.0, 0.5 * d * d, ad - 0.5)
    row_loc = jnp.sum(sl1 * pos)

    m = pred_ref[0, 4]
    for c in range(1, nc):
        m = jnp.maximum(m, pred_ref[0, 4 + c])
    s = zero
    for c in range(nc):
        s = s + jnp.exp(pred_ref[0, 4 + c] - m)
    lse = jnp.log(s) + m
    cls_i = cls_t.astype(jnp.int32)
    picked = zero
    for c in range(nc):
        picked = jnp.where(cls_i == c, pred_ref[0, 4 + c], picked)
    ce = lse - picked
    row_pos_loss = jnp.sum(ce * pos)
    ce0 = lse - pred_ref[0, 4]

    an = ce * (1.0 - pos)
    bits = jax.lax.bitcast_convert_type(an, jnp.int32)
    bits = jnp.where(valid, bits, -1)
    kf = jnp.minimum(_NEG_RATIO * np_row, float(nb - 1))

    def cgt(t):
        return jnp.sum(jnp.where(bits > t, 1, 0)).astype(f32)

    def vstep(_, lh):
        lo, hi = lh
        mid = lo + (hi - lo) // 2
        big = cgt(mid) >= kf
        return (jnp.where(big, mid, lo), jnp.where(big, hi, mid))

    lo0 = jnp.int32(-1)
    hi0 = jnp.int32(0x7F800000)
    _, thr = jax.lax.fori_loop(0, 32, vstep, (lo0, hi0))
    n_gt = cgt(thr)
    r_need = kf - n_gt

    eq = bits == thr

    def ceq_lt(mm):
        return jnp.sum(jnp.where(jnp.logical_and(eq, ids < mm), 1, 0)).astype(f32)

    def istep(_, lh):
        lo, hi = lh
        mid = lo + (hi - lo) // 2
        enough = ceq_lt(mid) >= r_need
        return (jnp.where(enough, lo, mid), jnp.where(enough, mid, hi))

    _, m_tie = jax.lax.fori_loop(0, 15, istep, (jnp.int32(0), jnp.int32(_NBP)))
    m_tie = jnp.where(r_need > 0.5, m_tie, 0)

    neg = jnp.logical_or(bits > thr, jnp.logical_and(eq, ids < m_tie))
    row_neg_loss = jnp.sum(ce0 * neg.astype(f32))

    @pl.when(r == 0)
    def _init():
        acc_ref[0] = 0.0
        acc_ref[1] = 0.0
        acc_ref[2] = 0.0

    acc_ref[0] += row_loc
    acc_ref[1] += row_pos_loss + row_neg_loss
    acc_ref[2] += np_row

    @pl.when(r == nbatch - 1)
    def _fin():
        out_ref[0, 0] = (acc_ref[0] + _ALPHA * acc_ref[1]) / acc_ref[2]


def kernel(pred, gt_boxes, gt_classes, anchors):
    b, nb, cp4 = pred.shape
    nc = cp4 - 4
    pred_t = jnp.transpose(pred, (0, 2, 1))
    pred_t = jnp.pad(pred_t, ((0, 0), (0, 0), (0, _NBP - nb)))
    pred_t = pred_t.reshape(b, cp4, _R, _L)
    anc_t = jnp.pad(anchors.T, ((0, 0), (0, _NBP - nb))).reshape(4, _R, _L)
    gtc = gt_classes.reshape(b, 1, 8)

    body = functools.partial(_row_body, nb=nb, nc=nc, nbatch=b)
    out = pl.pallas_call(
        body,
        grid=(b,),
        in_specs=[
            pl.BlockSpec((1, cp4, _R, _L), lambda r: (r, 0, 0, 0)),
            pl.BlockSpec((4, _R, _L), lambda r: (0, 0, 0)),
            pl.BlockSpec((1, 8, 4), lambda r: (r, 0, 0), memory_space=pltpu.SMEM),
            pl.BlockSpec((1, 1, 8), lambda r: (r, 0, 0), memory_space=pltpu.SMEM),
        ],
        out_specs=pl.BlockSpec((1, 1), lambda r: (0, 0), memory_space=pltpu.SMEM),
        out_shape=jax.ShapeDtypeStruct((1, 1), jnp.float32),
        scratch_shapes=[pltpu.SMEM((3,), jnp.float32)],
    )(pred_t, anc_t, gt_boxes, gtc)
    return out[0, 0]


# speedup vs baseline: 17.9101x; 1.4874x over previous
import functools

import jax
import jax.numpy as jnp
from jax import lax
from jax.experimental import pallas as pl
from jax.experimental.pallas import tpu as pltpu
from jax.experimental.pallas import tpu_sc as plsc

_NEG_RATIO = 3.0
_IOU_THR = 0.5
_ALPHA = 1.0

_R = 72
_L = 128
_NBP = _R * _L
_NV = _NBP // 16


def _row_body(pred_ref, anc_ref, gtb_ref, gtc_ref, bits_ref, ce0_ref, part_ref,
              *, nb, nc):
    f32 = jnp.float32

    ax = anc_ref[0]
    ay = anc_ref[1]
    aw = anc_ref[2]
    ah = anc_ref[3]
    ax1 = ax - aw / 2
    ay1 = ay - ah / 2
    ax2 = ax + aw / 2
    ay2 = ay + ah / 2
    area_a = (ax2 - ax1) * (ay2 - ay1)

    row_ids = jax.lax.broadcasted_iota(jnp.int32, (_R, _L), 0)
    lane_ids = jax.lax.broadcasted_iota(jnp.int32, (_R, _L), 1)
    ids = row_ids * _L + lane_ids
    valid = ids < nb

    g_cx = [gtb_ref[0, g, 0] for g in range(8)]
    g_cy = [gtb_ref[0, g, 1] for g in range(8)]
    g_w = [gtb_ref[0, g, 2] for g in range(8)]
    g_h = [gtb_ref[0, g, 3] for g in range(8)]
    g_cls = [(gtc_ref[0, 0, g] + 1).astype(f32) for g in range(8)]

    best_ov = None
    best_idx = None
    bp = []
    for g in range(8):
        gx1 = g_cx[g] - g_w[g] / 2
        gy1 = g_cy[g] - g_h[g] / 2
        gx2 = g_cx[g] + g_w[g] / 2
        gy2 = g_cy[g] + g_h[g] / 2
        garea = (gx2 - gx1) * (gy2 - gy1)
        iw = jnp.maximum(jnp.minimum(gx2, ax2) - jnp.maximum(gx1, ax1), 0.0)
        ih = jnp.maximum(jnp.minimum(gy2, ay2) - jnp.maximum(gy1, ay1), 0.0)
        inter = iw * ih
        iou = inter / (garea + area_a - inter + 1e-12)
        if g == 0:
            best_ov = iou
            best_idx = jnp.zeros((_R, _L), jnp.int32)
        else:
            upd = iou > best_ov
            best_ov = jnp.where(upd, iou, best_ov)
            best_idx = jnp.where(upd, g, best_idx)
        m = jnp.max(iou)
        cand = jnp.where(iou == m, ids, _NBP)
        bp.append(jnp.min(cand))

    def pick(vals):
        v = jnp.full((_R, _L), vals[0], f32)
        for g in range(1, 8):
            v = jnp.where(best_idx == g, vals[g], v)
        return v

    cx_p = pick(g_cx)
    cy_p = pick(g_cy)
    w_p = pick(g_w)
    h_p = pick(g_h)
    cls_p = pick(g_cls)

    over = best_ov > _IOU_THR
    match = over
    for g in range(8):
        fm = ids == bp[g]
        cx_p = jnp.where(fm, g_cx[g], cx_p)
        cy_p = jnp.where(fm, g_cy[g], cy_p)
        w_p = jnp.where(fm, g_w[g], w_p)
        h_p = jnp.where(fm, g_h[g], h_p)
        cls_p = jnp.where(fm, g_cls[g], cls_p)
        match = jnp.logical_or(match, fm)

    law = jnp.log(aw)
    lah = jnp.log(ah)
    w_safe = jnp.where(match, w_p, 1.0)
    h_safe = jnp.where(match, h_p, 1.0)
    zero = jnp.zeros((_R, _L), f32)
    t0 = jnp.where(match, (cx_p - ax) / aw, zero)
    t1 = jnp.where(match, (cy_p - ay) / ah, zero)
    t2 = jnp.where(match, jnp.log(w_safe) - law, zero)
    t3 = jnp.where(match, jnp.log(h_safe) - lah, zero)
    cls_t = jnp.where(match, cls_p, zero)

    pos = (cls_t > 0).astype(f32)
    np_row = jnp.sum(pos)

    locs = (t0, t1, t2, t3)
    sl1 = zero
    for c in range(4):
        d = pred_ref[0, c] - locs[c]
        ad = jnp.abs(d)
        sl1 = sl1 + jnp.where(ad < 1.0, 0.5 * d * d, ad - 0.5)
    row_loc = jnp.sum(sl1 * pos)

    m = pred_ref[0, 4]
    for c in range(1, nc):
        m = jnp.maximum(m, pred_ref[0, 4 + c])
    s = zero
    for c in range(nc):
        s = s + jnp.exp(pred_ref[0, 4 + c] - m)
    lse = jnp.log(s) + m
    cls_i = cls_t.astype(jnp.int32)
    picked = zero
    for c in range(nc):
        picked = jnp.where(cls_i == c, pred_ref[0, 4 + c], picked)
    ce = lse - picked
    row_pos_loss = jnp.sum(ce * pos)
    ce0 = lse - pred_ref[0, 4]

    an = ce * (1.0 - pos)
    bits = jax.lax.bitcast_convert_type(an, jnp.int32)
    bits_ref[0] = jnp.where(valid, bits, -1)
    ce0_ref[0] = ce0
    part_ref[0, 0, 0] = row_loc
    part_ref[0, 0, 1] = row_pos_loss
    part_ref[0, 0, 2] = np_row
    part_ref[0, 0, 3] = jnp.minimum(_NEG_RATIO * np_row, float(nb - 1))


def _tc_stage(pred_t, anc_t, gt_boxes, gtc, nb, nc, b):
    body = functools.partial(_row_body, nb=nb, nc=nc)
    return pl.pallas_call(
        body,
        grid=(b,),
        in_specs=[
            pl.BlockSpec((1, nc + 4, _R, _L), lambda r: (r, 0, 0, 0)),
            pl.BlockSpec((4, _R, _L), lambda r: (0, 0, 0)),
            pl.BlockSpec((1, 8, 4), lambda r: (r, 0, 0), memory_space=pltpu.SMEM),
            pl.BlockSpec((1, 1, 8), lambda r: (r, 0, 0), memory_space=pltpu.SMEM),
        ],
        out_specs=[
            pl.BlockSpec((1, _R, _L), lambda r: (r, 0, 0)),
            pl.BlockSpec((1, _R, _L), lambda r: (r, 0, 0)),
            pl.BlockSpec((1, 1, 4), lambda r: (r, 0, 0), memory_space=pltpu.SMEM),
        ],
        out_shape=[
            jax.ShapeDtypeStruct((b, _R, _L), jnp.int32),
            jax.ShapeDtypeStruct((b, _R, _L), jnp.float32),
            jax.ShapeDtypeStruct((b, 1, 4), jnp.float32),
        ],
    )(pred_t, anc_t, gt_boxes, gtc)


def _sc_mine(bits, ce0, kf, b):
    mesh = plsc.VectorSubcoreMesh(core_axis_name="c", subcore_axis_name="s")

    @functools.partial(
        pl.kernel,
        mesh=mesh,
        out_type=jax.ShapeDtypeStruct((b, 16), jnp.float32),
        compiler_params=pltpu.CompilerParams(needs_layout_passes=False),
        scratch_types=[
            pltpu.VMEM((_NBP,), jnp.int32),
            pltpu.VMEM((_NBP,), jnp.float32),
            pltpu.VMEM((256,), jnp.int32),
            pltpu.VMEM((16,), jnp.float32),
            pltpu.VMEM((16,), jnp.float32),
        ],
    )
    def k_fn(bits_h, ce0_h, kf_h, out_h, bits_v, ce0_v, hist_v, kf_v, out_v):
        cc = lax.axis_index("c")
        ss = lax.axis_index("s")
        w = ss * 2 + cc

        @pl.when(w < b)
        def _():
            pltpu.sync_copy(bits_h.at[w], bits_v)
            pltpu.sync_copy(ce0_h.at[w], ce0_v)
            pltpu.sync_copy(kf_h, kf_v)
            lane = lax.iota(jnp.int32, 16)
            k_i = jnp.sum(jnp.where(lane == w, kf_v[...], 0.0)).astype(jnp.int32)

            zeros16 = jnp.zeros((16,), jnp.int32)
            ones16 = jnp.ones((16,), jnp.int32)

            prefix = jnp.int32(0)
            kk = k_i
            for p in (24, 16, 8, 0):
                def zbody(i, x):
                    hist_v[pl.ds(i * 16, 16)] = zeros16
                    return x

                lax.fori_loop(0, 16, zbody, 0)

                shift_hi = p + 8

                def hbody(i, pref):
                    v = bits_v[pl.ds(i * 16, 16)]
                    ok = v >= 0
                    if p < 24:
                        ok = jnp.logical_and(ok, (v >> shift_hi) == pref)
                    bb = (v >> p) & 0xFF
                    plsc.addupdate_scatter(hist_v, [bb], ones16, mask=ok)
                    return pref

                prefix = lax.fori_loop(0, _NV, hbody, prefix)

                def sbody(i, carry):
                    suf, nsel = carry
                    j = 15 - i
                    h = hist_v[pl.ds(j * 16, 16)]
                    rc = lax.rev(jnp.cumsum(lax.rev(h, (0,))), (0,)) + suf
                    nsel = nsel + jnp.sum(jnp.where(rc >= kk, 1, 0))
                    suf = suf + jnp.sum(h)
                    return suf, nsel

                _, nsel = lax.fori_loop(0, 16, sbody,
                                        (jnp.int32(0), jnp.int32(0)))
                t = nsel - 1

                def gbody(i, acc):
                    ids16 = lane + i * 16
                    h = hist_v[pl.ds(i * 16, 16)]
                    return acc + jnp.sum(jnp.where(ids16 > t, h, 0))

                n_gt = lax.fori_loop(0, 16, gbody, jnp.int32(0))
                kk = kk - n_gt
                prefix = jnp.bitwise_or(prefix << 8, t)

            thr = jnp.where(k_i > 0, prefix, jnp.int32(0x7F800000))
            r_need = jnp.where(k_i > 0, kk, 0)

            def fbody(i, carry):
                cnt_eq, accv = carry
                v = bits_v[pl.ds(i * 16, 16)]
                gt = v > thr
                eq = v == thr
                pc = jnp.cumsum(eq.astype(jnp.int32))
                sel = jnp.logical_or(gt, jnp.logical_and(eq, (cnt_eq + pc) <= r_need))
                cv = ce0_v[pl.ds(i * 16, 16)]
                accv = accv + jnp.where(sel, cv, 0.0)
                cnt_eq = cnt_eq + jnp.sum(eq.astype(jnp.int32))
                return cnt_eq, accv

            _, accv = lax.fori_loop(
                0, _NV, fbody, (jnp.int32(0), jnp.zeros((16,), jnp.float32))
            )
            out_v[...] = jnp.where(lane == 0, jnp.sum(accv), 0.0)
            pltpu.sync_copy(out_v, out_h.at[w])

    return k_fn(bits, ce0, kf)


def kernel(pred, gt_boxes, gt_classes, anchors):
    b, nb, cp4 = pred.shape
    nc = cp4 - 4
    pred_t = jnp.transpose(pred, (0, 2, 1))
    pred_t = jnp.pad(pred_t, ((0, 0), (0, 0), (0, _NBP - nb)))
    pred_t = pred_t.reshape(b, cp4, _R, _L)
    anc_t = jnp.pad(anchors.T, ((0, 0), (0, _NBP - nb))).reshape(4, _R, _L)
    gtc = gt_classes.reshape(b, 1, 8)

    bits, ce0, part = _tc_stage(pred_t, anc_t, gt_boxes, gtc, nb, nc, b)
    part = part.reshape(b, 4)
    neg = _sc_mine(bits.reshape(b, _NBP), ce0.reshape(b, _NBP), part[:, 3], b)
    cls_loss = jnp.sum(part[:, 1]) + jnp.sum(neg[:, 0])
    return (jnp.sum(part[:, 0]) + _ALPHA * cls_loss) / jnp.sum(part[:, 2])


# speedup vs baseline: 18.2824x; 1.0208x over previous
import functools

import jax
import jax.numpy as jnp
from jax import lax
from jax.experimental import pallas as pl
from jax.experimental.pallas import tpu as pltpu
from jax.experimental.pallas import tpu_sc as plsc

_NEG_RATIO = 3.0
_IOU_THR = 0.5
_ALPHA = 1.0

_R = 72
_L = 128
_NBP = _R * _L
_NV = _NBP // 16


def _row_body(pred_ref, anc_ref, gtb_ref, gtc_ref, bits_ref, ce0_ref, part_ref,
              *, nb, nc):
    f32 = jnp.float32

    ax = anc_ref[0]
    ay = anc_ref[1]
    aw = anc_ref[2]
    ah = anc_ref[3]
    ax1 = ax - aw / 2
    ay1 = ay - ah / 2
    ax2 = ax + aw / 2
    ay2 = ay + ah / 2
    area_a = (ax2 - ax1) * (ay2 - ay1)

    row_ids = jax.lax.broadcasted_iota(jnp.int32, (_R, _L), 0)
    lane_ids = jax.lax.broadcasted_iota(jnp.int32, (_R, _L), 1)
    ids = row_ids * _L + lane_ids
    valid = ids < nb

    g_cx = [gtb_ref[0, g, 0] for g in range(8)]
    g_cy = [gtb_ref[0, g, 1] for g in range(8)]
    g_w = [gtb_ref[0, g, 2] for g in range(8)]
    g_h = [gtb_ref[0, g, 3] for g in range(8)]
    g_cls = [(gtc_ref[0, 0, g] + 1).astype(f32) for g in range(8)]

    best_ov = None
    best_idx = None
    bp = []
    for g in range(8):
        gx1 = g_cx[g] - g_w[g] / 2
        gy1 = g_cy[g] - g_h[g] / 2
        gx2 = g_cx[g] + g_w[g] / 2
        gy2 = g_cy[g] + g_h[g] / 2
        garea = (gx2 - gx1) * (gy2 - gy1)
        iw = jnp.maximum(jnp.minimum(gx2, ax2) - jnp.maximum(gx1, ax1), 0.0)
        ih = jnp.maximum(jnp.minimum(gy2, ay2) - jnp.maximum(gy1, ay1), 0.0)
        inter = iw * ih
        iou = inter / (garea + area_a - inter + 1e-12)
        if g == 0:
            best_ov = iou
            best_idx = jnp.zeros((_R, _L), jnp.int32)
        else:
            upd = iou > best_ov
            best_ov = jnp.where(upd, iou, best_ov)
            best_idx = jnp.where(upd, g, best_idx)
        m = jnp.max(iou)
        cand = jnp.where(iou == m, ids, _NBP)
        bp.append(jnp.min(cand))

    def pick(vals):
        v = jnp.full((_R, _L), vals[0], f32)
        for g in range(1, 8):
            v = jnp.where(best_idx == g, vals[g], v)
        return v

    cx_p = pick(g_cx)
    cy_p = pick(g_cy)
    w_p = pick(g_w)
    h_p = pick(g_h)
    cls_p = pick(g_cls)

    over = best_ov > _IOU_THR
    match = over
    for g in range(8):
        fm = ids == bp[g]
        cx_p = jnp.where(fm, g_cx[g], cx_p)
        cy_p = jnp.where(fm, g_cy[g], cy_p)
        w_p = jnp.where(fm, g_w[g], w_p)
        h_p = jnp.where(fm, g_h[g], h_p)
        cls_p = jnp.where(fm, g_cls[g], cls_p)
        match = jnp.logical_or(match, fm)

    law = jnp.log(aw)
    lah = jnp.log(ah)
    w_safe = jnp.where(match, w_p, 1.0)
    h_safe = jnp.where(match, h_p, 1.0)
    zero = jnp.zeros((_R, _L), f32)
    t0 = jnp.where(match, (cx_p - ax) / aw, zero)
    t1 = jnp.where(match, (cy_p - ay) / ah, zero)
    t2 = jnp.where(match, jnp.log(w_safe) - law, zero)
    t3 = jnp.where(match, jnp.log(h_safe) - lah, zero)
    cls_t = jnp.where(match, cls_p, zero)

    pos = (cls_t > 0).astype(f32)
    np_row = jnp.sum(pos)

    locs = (t0, t1, t2, t3)
    sl1 = zero
    for c in range(4):
        d = pred_ref[0, c] - locs[c]
        ad = jnp.abs(d)
        sl1 = sl1 + jnp.where(ad < 1.0, 0.5 * d * d, ad - 0.5)
    row_loc = jnp.sum(sl1 * pos)

    m = pred_ref[0, 4]
    for c in range(1, nc):
        m = jnp.maximum(m, pred_ref[0, 4 + c])
    s = zero
    for c in range(nc):
        s = s + jnp.exp(pred_ref[0, 4 + c] - m)
    lse = jnp.log(s) + m
    cls_i = cls_t.astype(jnp.int32)
    picked = zero
    for c in range(nc):
        picked = jnp.where(cls_i == c, pred_ref[0, 4 + c], picked)
    ce = lse - picked
    row_pos_loss = jnp.sum(ce * pos)
    ce0 = lse - pred_ref[0, 4]

    an = ce * (1.0 - pos)
    bits = jax.lax.bitcast_convert_type(an, jnp.int32)
    bits_ref[0] = jnp.where(valid, bits, -1)
    ce0_ref[0] = ce0
    part_ref[0, 0, 0] = row_loc
    part_ref[0, 0, 1] = row_pos_loss
    part_ref[0, 0, 2] = np_row
    part_ref[0, 0, 3] = jnp.minimum(_NEG_RATIO * np_row, float(nb - 1))


def _tc_stage(pred_t, anc_t, gt_boxes, gtc, nb, nc, b):
    body = functools.partial(_row_body, nb=nb, nc=nc)
    return pl.pallas_call(
        body,
        grid=(b,),
        in_specs=[
            pl.BlockSpec((1, nc + 4, _R, _L), lambda r: (r, 0, 0, 0)),
            pl.BlockSpec((4, _R, _L), lambda r: (0, 0, 0)),
            pl.BlockSpec((1, 8, 4), lambda r: (r, 0, 0), memory_space=pltpu.SMEM),
            pl.BlockSpec((1, 1, 8), lambda r: (r, 0, 0), memory_space=pltpu.SMEM),
        ],
        out_specs=[
            pl.BlockSpec((1, _R, _L), lambda r: (r, 0, 0)),
            pl.BlockSpec((1, _R, _L), lambda r: (r, 0, 0)),
            pl.BlockSpec((1, 1, 4), lambda r: (r, 0, 0), memory_space=pltpu.SMEM),
        ],
        out_shape=[
            jax.ShapeDtypeStruct((b, _R, _L), jnp.int32),
            jax.ShapeDtypeStruct((b, _R, _L), jnp.float32),
            jax.ShapeDtypeStruct((b, 1, 4), jnp.float32),
        ],
    )(pred_t, anc_t, gt_boxes, gtc)


def _sc_mine(bits, ce0, kf, b):
    mesh = plsc.VectorSubcoreMesh(core_axis_name="c", subcore_axis_name="s")

    @functools.partial(
        pl.kernel,
        mesh=mesh,
        out_type=jax.ShapeDtypeStruct((b, 16), jnp.float32),
        compiler_params=pltpu.CompilerParams(needs_layout_passes=False),
        scratch_types=[
            pltpu.VMEM((_NBP,), jnp.int32),
            pltpu.VMEM((_NBP,), jnp.float32),
            pltpu.VMEM((256,), jnp.int32),
            pltpu.VMEM((16,), jnp.float32),
            pltpu.VMEM((16,), jnp.float32),
        ],
    )
    def k_fn(bits_h, ce0_h, kf_h, out_h, bits_v, ce0_v, hist_v, kf_v, out_v):
        cc = lax.axis_index("c")
        ss = lax.axis_index("s")
        w = ss * 2 + cc

        @pl.when(w < b)
        def _():
            pltpu.sync_copy(bits_h.at[w], bits_v)
            pltpu.sync_copy(ce0_h.at[w], ce0_v)
            pltpu.sync_copy(kf_h, kf_v)
            lane = lax.iota(jnp.int32, 16)
            k_i = jnp.sum(jnp.where(lane == w, kf_v[...], 0.0)).astype(jnp.int32)

            zeros16 = jnp.zeros((16,), jnp.int32)
            ones16 = jnp.ones((16,), jnp.int32)

            prefix = jnp.int32(0)
            kk = k_i
            for p in (24, 16, 8, 0):
                def zbody(i, x):
                    hist_v[pl.ds(i * 16, 16)] = zeros16
                    return x

                lax.fori_loop(0, 16, zbody, 0)

                shift_hi = p + 8

                def hbody(i, pref):
                    for u in range(4):
                        v = bits_v[pl.ds(i * 64 + u * 16, 16)]
                        if p == 24:
                            ok = v >= 0
                        else:
                            ok = (v >> shift_hi) == pref
                        bb = (v >> p) & 0xFF
                        plsc.addupdate_scatter(hist_v, [bb], ones16, mask=ok)
                    return pref

                prefix = lax.fori_loop(0, _NV // 4, hbody, prefix)

                def sbody(i, carry):
                    suf, nsel = carry
                    j = 15 - i
                    h = hist_v[pl.ds(j * 16, 16)]
                    rc = lax.rev(jnp.cumsum(lax.rev(h, (0,))), (0,)) + suf
                    nsel = nsel + jnp.sum(jnp.where(rc >= kk, 1, 0))
                    suf = suf + jnp.sum(h)
                    return suf, nsel

                _, nsel = lax.fori_loop(0, 16, sbody,
                                        (jnp.int32(0), jnp.int32(0)))
                t = nsel - 1

                def gbody(i, acc):
                    ids16 = lane + i * 16
                    h = hist_v[pl.ds(i * 16, 16)]
                    return acc + jnp.sum(jnp.where(ids16 > t, h, 0))

                n_gt = lax.fori_loop(0, 16, gbody, jnp.int32(0))
                kk = kk - n_gt
                prefix = jnp.bitwise_or(prefix << 8, t)

            thr = jnp.where(k_i > 0, prefix, jnp.int32(0x7F800000))
            r_need = jnp.where(k_i > 0, kk, 0)

            def fbody(i, carry):
                cnt_eq, accv = carry
                for u in range(2):
                    v = bits_v[pl.ds(i * 32 + u * 16, 16)]
                    gt = v > thr
                    eq = v == thr
                    pc = jnp.cumsum(eq.astype(jnp.int32))
                    sel = jnp.logical_or(
                        gt, jnp.logical_and(eq, (cnt_eq + pc) <= r_need)
                    )
                    cv = ce0_v[pl.ds(i * 32 + u * 16, 16)]
                    accv = accv + jnp.where(sel, cv, 0.0)
                    cnt_eq = cnt_eq + jnp.sum(eq.astype(jnp.int32))
                return cnt_eq, accv

            _, accv = lax.fori_loop(
                0, _NV // 2, fbody, (jnp.int32(0), jnp.zeros((16,), jnp.float32))
            )
            out_v[...] = jnp.where(lane == 0, jnp.sum(accv), 0.0)
            pltpu.sync_copy(out_v, out_h.at[w])

    return k_fn(bits, ce0, kf)


def kernel(pred, gt_boxes, gt_classes, anchors):
    b, nb, cp4 = pred.shape
    nc = cp4 - 4
    pred_t = jnp.transpose(pred, (0, 2, 1))
    pred_t = jnp.pad(pred_t, ((0, 0), (0, 0), (0, _NBP - nb)))
    pred_t = pred_t.reshape(b, cp4, _R, _L)
    anc_t = jnp.pad(anchors.T, ((0, 0), (0, _NBP - nb))).reshape(4, _R, _L)
    gtc = gt_classes.reshape(b, 1, 8)

    bits, ce0, part = _tc_stage(pred_t, anc_t, gt_boxes, gtc, nb, nc, b)
    part = part.reshape(b, 4)
    neg = _sc_mine(bits.reshape(b, _NBP), ce0.reshape(b, _NBP), part[:, 3], b)
    cls_loss = jnp.sum(part[:, 1]) + jnp.sum(neg[:, 0])
    return (jnp.sum(part[:, 0]) + _ALPHA * cls_loss) / jnp.sum(part[:, 2])


# speedup vs baseline: 18.7808x; 1.0273x over previous
import functools

import jax
import jax.numpy as jnp
from jax import lax
from jax.experimental import pallas as pl
from jax.experimental.pallas import tpu as pltpu
from jax.experimental.pallas import tpu_sc as plsc

_NEG_RATIO = 3.0
_IOU_THR = 0.5
_ALPHA = 1.0

_R = 72
_L = 128
_NBP = _R * _L
_NV = _NBP // 16


def _row_body(pred_ref, anc_ref, gtb_ref, gtc_ref, bits_ref, ce0_ref, part_ref,
              *, nb, nc):
    f32 = jnp.float32

    ax = anc_ref[0]
    ay = anc_ref[1]
    aw = anc_ref[2]
    ah = anc_ref[3]
    ax1 = ax - aw / 2
    ay1 = ay - ah / 2
    ax2 = ax + aw / 2
    ay2 = ay + ah / 2
    area_a = (ax2 - ax1) * (ay2 - ay1)

    row_ids = jax.lax.broadcasted_iota(jnp.int32, (_R, _L), 0)
    lane_ids = jax.lax.broadcasted_iota(jnp.int32, (_R, _L), 1)
    ids = row_ids * _L + lane_ids
    valid = ids < nb

    g_cx = [gtb_ref[0, g, 0] for g in range(8)]
    g_cy = [gtb_ref[0, g, 1] for g in range(8)]
    g_w = [gtb_ref[0, g, 2] for g in range(8)]
    g_h = [gtb_ref[0, g, 3] for g in range(8)]
    g_cls = [(gtc_ref[0, 0, g] + 1).astype(f32) for g in range(8)]

    best_ov = None
    best_idx = None
    bp = []
    for g in range(8):
        gx1 = g_cx[g] - g_w[g] / 2
        gy1 = g_cy[g] - g_h[g] / 2
        gx2 = g_cx[g] + g_w[g] / 2
        gy2 = g_cy[g] + g_h[g] / 2
        garea = (gx2 - gx1) * (gy2 - gy1)
        iw = jnp.maximum(jnp.minimum(gx2, ax2) - jnp.maximum(gx1, ax1), 0.0)
        ih = jnp.maximum(jnp.minimum(gy2, ay2) - jnp.maximum(gy1, ay1), 0.0)
        inter = iw * ih
        iou = inter / (garea + area_a - inter + 1e-12)
        if g == 0:
            best_ov = iou
            best_idx = jnp.zeros((_R, _L), jnp.int32)
        else:
            upd = iou > best_ov
            best_ov = jnp.where(upd, iou, best_ov)
            best_idx = jnp.where(upd, g, best_idx)
        m = jnp.max(iou)
        cand = jnp.where(iou == m, ids, _NBP)
        bp.append(jnp.min(cand))

    def pick(vals):
        v = jnp.full((_R, _L), vals[0], f32)
        for g in range(1, 8):
            v = jnp.where(best_idx == g, vals[g], v)
        return v

    cx_p = pick(g_cx)
    cy_p = pick(g_cy)
    w_p = pick(g_w)
    h_p = pick(g_h)
    cls_p = pick(g_cls)

    over = best_ov > _IOU_THR
    match = over
    for g in range(8):
        fm = ids == bp[g]
        cx_p = jnp.where(fm, g_cx[g], cx_p)
        cy_p = jnp.where(fm, g_cy[g], cy_p)
        w_p = jnp.where(fm, g_w[g], w_p)
        h_p = jnp.where(fm, g_h[g], h_p)
        cls_p = jnp.where(fm, g_cls[g], cls_p)
        match = jnp.logical_or(match, fm)

    law = jnp.log(aw)
    lah = jnp.log(ah)
    w_safe = jnp.where(match, w_p, 1.0)
    h_safe = jnp.where(match, h_p, 1.0)
    zero = jnp.zeros((_R, _L), f32)
    t0 = jnp.where(match, (cx_p - ax) / aw, zero)
    t1 = jnp.where(match, (cy_p - ay) / ah, zero)
    t2 = jnp.where(match, jnp.log(w_safe) - law, zero)
    t3 = jnp.where(match, jnp.log(h_safe) - lah, zero)
    cls_t = jnp.where(match, cls_p, zero)

    pos = (cls_t > 0).astype(f32)
    np_row = jnp.sum(pos)

    locs = (t0, t1, t2, t3)
    sl1 = zero
    for c in range(4):
        d = pred_ref[0, c] - locs[c]
        ad = jnp.abs(d)
        sl1 = sl1 + jnp.where(ad < 1.0, 0.5 * d * d, ad - 0.5)
    row_loc = jnp.sum(sl1 * pos)

    m = pred_ref[0, 4]
    for c in range(1, nc):
        m = jnp.maximum(m, pred_ref[0, 4 + c])
    s = zero
    for c in range(nc):
        s = s + jnp.exp(pred_ref[0, 4 + c] - m)
    lse = jnp.log(s) + m
    cls_i = cls_t.astype(jnp.int32)
    picked = zero
    for c in range(nc):
        picked = jnp.where(cls_i == c, pred_ref[0, 4 + c], picked)
    ce = lse - picked
    row_pos_loss = jnp.sum(ce * pos)
    ce0 = lse - pred_ref[0, 4]

    an = ce * (1.0 - pos)
    bits = jax.lax.bitcast_convert_type(an, jnp.int32)
    bits_ref[0] = jnp.where(valid, bits, -1)
    ce0_ref[0] = ce0
    part_ref[0, 0, 0] = row_loc
    part_ref[0, 0, 1] = row_pos_loss
    part_ref[0, 0, 2] = np_row
    part_ref[0, 0, 3] = jnp.minimum(_NEG_RATIO * np_row, float(nb - 1))


def _tc_stage(pred_t, anc_t, gt_boxes, gtc, nb, nc, b):
    body = functools.partial(_row_body, nb=nb, nc=nc)
    return pl.pallas_call(
        body,
        grid=(b,),
        in_specs=[
            pl.BlockSpec((1, nc + 4, _R, _L), lambda r: (r, 0, 0, 0)),
            pl.BlockSpec((4, _R, _L), lambda r: (0, 0, 0)),
            pl.BlockSpec((1, 8, 4), lambda r: (r, 0, 0), memory_space=pltpu.SMEM),
            pl.BlockSpec((1, 1, 8), lambda r: (r, 0, 0), memory_space=pltpu.SMEM),
        ],
        out_specs=[
            pl.BlockSpec((1, _R, _L), lambda r: (r, 0, 0)),
            pl.BlockSpec((1, _R, _L), lambda r: (r, 0, 0)),
            pl.BlockSpec((1, 1, 4), lambda r: (r, 0, 0), memory_space=pltpu.SMEM),
        ],
        out_shape=[
            jax.ShapeDtypeStruct((b, _R, _L), jnp.int32),
            jax.ShapeDtypeStruct((b, _R, _L), jnp.float32),
            jax.ShapeDtypeStruct((b, 1, 4), jnp.float32),
        ],
    )(pred_t, anc_t, gt_boxes, gtc)


def _sc_mine(bits, ce0, part_t, b):
    mesh = plsc.VectorSubcoreMesh(core_axis_name="c", subcore_axis_name="s")

    @functools.partial(
        pl.kernel,
        mesh=mesh,
        out_type=jax.ShapeDtypeStruct((16,), jnp.float32),
        compiler_params=pltpu.CompilerParams(needs_layout_passes=False),
        scratch_types=[
            pltpu.VMEM((_NBP,), jnp.int32),
            pltpu.VMEM((_NBP,), jnp.float32),
            pltpu.VMEM((256,), jnp.int32),
            pltpu.VMEM((16,), jnp.float32),
            pltpu.VMEM((16,), jnp.float32),
            pltpu.VMEM((256,), jnp.float32),
            pltpu.VMEM_SHARED((256,), jnp.float32),
        ],
    )
    def k_fn(bits_h, ce0_h, part_h, out_h, bits_v, ce0_v, hist_v, kf_v, out_v,
             stage_v, shared):
        cc = lax.axis_index("c")
        ss = lax.axis_index("s")
        w = ss

        @pl.when(cc == 0)
        def _():
            pltpu.sync_copy(bits_h.at[w], bits_v)
            pltpu.sync_copy(ce0_h.at[w], ce0_v)
            pltpu.sync_copy(part_h.at[3], kf_v)
            lane = lax.iota(jnp.int32, 16)
            k_i = jnp.sum(jnp.where(lane == w, kf_v[...], 0.0)).astype(jnp.int32)

            zeros16 = jnp.zeros((16,), jnp.int32)
            ones16 = jnp.ones((16,), jnp.int32)

            prefix = jnp.int32(0)
            kk = k_i
            for p in (24, 16, 8, 0):
                def zbody(i, x):
                    hist_v[pl.ds(i * 16, 16)] = zeros16
                    return x

                lax.fori_loop(0, 16, zbody, 0)

                shift_hi = p + 8

                def hbody(i, pref):
                    for u in range(4):
                        v = bits_v[pl.ds(i * 64 + u * 16, 16)]
                        if p == 24:
                            ok = v >= 0
                        else:
                            ok = (v >> shift_hi) == pref
                        bb = (v >> p) & 0xFF
                        plsc.addupdate_scatter(hist_v, [bb], ones16, mask=ok)
                    return pref

                prefix = lax.fori_loop(0, _NV // 4, hbody, prefix)

                def sbody(i, carry):
                    suf, nsel = carry
                    j = 15 - i
                    h = hist_v[pl.ds(j * 16, 16)]
                    rc = lax.rev(jnp.cumsum(lax.rev(h, (0,))), (0,)) + suf
                    nsel = nsel + jnp.sum(jnp.where(rc >= kk, 1, 0))
                    suf = suf + jnp.sum(h)
                    return suf, nsel

                _, nsel = lax.fori_loop(0, 16, sbody,
                                        (jnp.int32(0), jnp.int32(0)))
                t = nsel - 1

                def gbody(i, acc):
                    ids16 = lane + i * 16
                    h = hist_v[pl.ds(i * 16, 16)]
                    return acc + jnp.sum(jnp.where(ids16 > t, h, 0))

                n_gt = lax.fori_loop(0, 16, gbody, jnp.int32(0))
                kk = kk - n_gt
                prefix = jnp.bitwise_or(prefix << 8, t)

            thr = jnp.where(k_i > 0, prefix, jnp.int32(0x7F800000))
            r_need = jnp.where(k_i > 0, kk, 0)

            def fbody(i, carry):
                cnt_eq, accv = carry
                for u in range(2):
                    v = bits_v[pl.ds(i * 32 + u * 16, 16)]
                    gt = v > thr
                    eq = v == thr
                    pc = jnp.cumsum(eq.astype(jnp.int32))
                    sel = jnp.logical_or(
                        gt, jnp.logical_and(eq, (cnt_eq + pc) <= r_need)
                    )
                    cv = ce0_v[pl.ds(i * 32 + u * 16, 16)]
                    accv = accv + jnp.where(sel, cv, 0.0)
                    cnt_eq = cnt_eq + jnp.sum(eq.astype(jnp.int32))
                return cnt_eq, accv

            _, accv = lax.fori_loop(
                0, _NV // 2, fbody, (jnp.int32(0), jnp.zeros((16,), jnp.float32))
            )
            out_v[...] = jnp.where(lane == 0, jnp.sum(accv), 0.0)
            pltpu.sync_copy(out_v, shared.at[pl.ds(w * 16, 16)])
            plsc.subcore_barrier()

            @pl.when(ss == 0)
            def _fin():
                pltpu.sync_copy(shared, stage_v)

                def rb(j, acc):
                    return acc + stage_v[pl.ds(j * 16, 16)]

                accn = lax.fori_loop(0, b, rb, jnp.zeros((16,), jnp.float32))
                neg_total = jnp.sum(accn)
                pltpu.sync_copy(part_h.at[0], kf_v)
                loc_total = jnp.sum(kf_v[...])
                pltpu.sync_copy(part_h.at[1], kf_v)
                pos_total = jnp.sum(kf_v[...])
                pltpu.sync_copy(part_h.at[2], kf_v)
                np_total = jnp.sum(kf_v[...])
                num_v = jnp.where(
                    lane == 0, loc_total + _ALPHA * (pos_total + neg_total), 0.0
                )
                den_v = jnp.where(lane == 0, np_total, 1.0)
                out_v[...] = num_v / den_v
                pltpu.sync_copy(out_v, out_h)

    return k_fn(bits, ce0, part_t)


def kernel(pred, gt_boxes, gt_classes, anchors):
    b, nb, cp4 = pred.shape
    nc = cp4 - 4
    pred_t = jnp.transpose(pred, (0, 2, 1))
    pred_t = jnp.pad(pred_t, ((0, 0), (0, 0), (0, _NBP - nb)))
    pred_t = pred_t.reshape(b, cp4, _R, _L)
    anc_t = jnp.pad(anchors.T, ((0, 0), (0, _NBP - nb))).reshape(4, _R, _L)
    gtc = gt_classes.reshape(b, 1, 8)

    bits, ce0, part = _tc_stage(pred_t, anc_t, gt_boxes, gtc, nb, nc, b)
    part_t = part.reshape(b, 4).T
    out = _sc_mine(bits.reshape(b, _NBP), ce0.reshape(b, _NBP), part_t, b)
    return out[0]
